# baseline (speedup 1.0000x reference)
import functools

import jax
import jax.numpy as jnp
from jax import lax
from jax.experimental import pallas as pl
from jax.experimental.pallas import tpu as pltpu
from jax.experimental.pallas import tpu_sc as plsc

B = 4096
V = 100000
D_SUB = 8
D_ENC = 8
C_IN = 16
L = 200
K = 6
H = 128
MAX_NORM = 5.0

_NC = 2
_NS = 16
_NW = _NC * _NS
_BPW = B // _NW

_RPG = 128 // D_SUB
_VG = V * D_SUB // 128

_BB = 128
_NB = B // _BB
_W = 2 * (K - 1) + 1


def _gather_rows(table_view, idxq):
    mesh = plsc.VectorSubcoreMesh(core_axis_name="c", subcore_axis_name="s")

    @functools.partial(
        pl.kernel,
        mesh=mesh,
        out_type=jax.ShapeDtypeStruct((B, 128), jnp.float32),
        scratch_types=[
            pltpu.VMEM((_BPW,), jnp.int32),
            pltpu.VMEM((_BPW, 128), jnp.float32),
            pltpu.SemaphoreType.DMA,
        ],
    )
    def k(table_hbm, idx_hbm, out_hbm, idx_v, rows_v, sem):
        wid = lax.axis_index("s") * _NC + lax.axis_index("c")
        base = wid * _BPW
        pltpu.sync_copy(idx_hbm.at[pl.ds(base, _BPW)], idx_v)
        pltpu.async_copy(table_hbm.at[idx_v], rows_v, sem).wait()
        pltpu.sync_copy(rows_v, out_hbm.at[pl.ds(base, _BPW)])

    return k(table_view, idxq)


def _encoder_body(pos_ref, phase_ref, gath_ref, hist_ref, w1f_ref, b1_ref,
                  w2f_ref, b2_ref, out_ref):
    X = hist_ref[...]
    pos = pos_ref[...]
    s = pos - (_W - 1)
    sp = s + 16
    mc = sp // 16
    r = sp - 16 * mc

    def c3(c):
        return c.reshape(_BB, 1, 1)

    zeros16 = jnp.zeros((_BB, C_IN, 16), jnp.float32)
    zeros8 = jnp.zeros((_BB, C_IN, 8), jnp.float32)
    acc = jnp.concatenate([zeros16, X[:, :, 0:16]], axis=2)
    for mm in range(1, 12):
        sl = X[:, :, 16 * mm - 16:16 * mm + 16]
        acc = jnp.where(c3(mc == mm), sl, acc)
    sl12 = jnp.concatenate([X[:, :, 176:200], zeros8], axis=2)
    Y = jnp.where(c3(mc == 12), sl12, acc)
    b8 = r >= 8
    r1 = r - 8 * b8.astype(jnp.int32)
    Z = jnp.where(c3(b8), Y[:, :, 8:32], Y[:, :, 0:24])
    b4 = r1 >= 4
    r2 = r1 - 4 * b4.astype(jnp.int32)
    Z = jnp.where(c3(b4), Z[:, :, 4:24], Z[:, :, 0:20])
    bt = r2 >= 2
    r3 = r2 - 2 * bt.astype(jnp.int32)
    Z = jnp.where(c3(bt), Z[:, :, 2:20], Z[:, :, 0:18])
    bo = r3 >= 1
    Z = jnp.where(c3(bo), Z[:, :, 1:18], Z[:, :, 0:17])
    xs = [Z[:, :, j] for j in range(_W)]
    w1f = w1f_ref[...]
    b1 = b1_ref[...]
    rs = []
    for t in range(K):
        patch = jnp.concatenate(xs[t:t + K], axis=1)
        r = jnp.dot(patch, w1f, preferred_element_type=jnp.float32) + b1
        r = jnp.maximum(r, 0.0)
        valid = (pos >= (K - 1) - t).astype(jnp.float32)
        rs.append(r * valid)
    h1 = jnp.concatenate(rs, axis=1)
    enc = jnp.dot(h1, w2f_ref[...], preferred_element_type=jnp.float32)
    enc = enc + b2_ref[...]
    G = gath_ref[...]
    ph = phase_ref[...]
    sub = jnp.zeros((_BB, D_SUB), jnp.float32)
    for c in range(_RPG):
        m = (ph == c).astype(jnp.float32)
        sub = sub + G[:, c * D_SUB:(c + 1) * D_SUB] * m
    n2 = jnp.sum(sub * sub, axis=1, keepdims=True)
    norm = jnp.sqrt(n2)
    scale = jnp.minimum(1.0, MAX_NORM / jnp.maximum(norm, 1e-7))
    out_ref[...] = jnp.concatenate([sub * scale, enc], axis=1)


def kernel(indices, history, history_lengths, table, w1, b1, w2, b2):
    idx = indices.astype(jnp.int32)
    pos2 = jnp.clip(history_lengths.astype(jnp.int32) - 1, 0, L - 1)
    pos2 = pos2.reshape(B, 1)
    table_view = table.reshape(_VG, 128)
    idxq = idx // _RPG
    phase2 = (idx % _RPG).reshape(B, 1)
    gath = _gather_rows(table_view, idxq)

    w1f = w1.transpose(2, 1, 0).reshape(K * C_IN, H)
    w2f = w2.transpose(2, 1, 0).reshape(K * H, D_ENC)
    b1r = b1.reshape(1, H)
    b2r = b2.reshape(1, D_ENC)

    out = pl.pallas_call(
        _encoder_body,
        grid=(_NB,),
        in_specs=[
            pl.BlockSpec((_BB, 1), lambda i: (i, 0)),
            pl.BlockSpec((_BB, 1), lambda i: (i, 0)),
            pl.BlockSpec((_BB, 128), lambda i: (i, 0)),
            pl.BlockSpec((_BB, C_IN, L), lambda i: (i, 0, 0)),
            pl.BlockSpec((K * C_IN, H), lambda i: (0, 0)),
            pl.BlockSpec((1, H), lambda i: (0, 0)),
            pl.BlockSpec((K * H, D_ENC), lambda i: (0, 0)),
            pl.BlockSpec((1, D_ENC), lambda i: (0, 0)),
        ],
        out_specs=pl.BlockSpec((_BB, D_SUB + D_ENC), lambda i: (i, 0)),
        out_shape=jax.ShapeDtypeStruct((B, D_SUB + D_ENC), jnp.float32),
    )(pos2, phase2, gath, history, w1f, b1r, w2f, b2r)
    return out


# speedup vs baseline: 1.7389x; 1.7389x over previous
import functools

import jax
import jax.numpy as jnp
from jax import lax
from jax.experimental import pallas as pl
from jax.experimental.pallas import tpu as pltpu
from jax.experimental.pallas import tpu_sc as plsc

B = 4096
V = 100000
D_SUB = 8
D_ENC = 8
C_IN = 16
L = 200
K = 6
H = 128
MAX_NORM = 5.0

_NC = 2
_NS = 16
_NW = _NC * _NS
_BPW = B // _NW

_RPG = 128 // D_SUB
_VG = V * D_SUB // 128

_BB = 128
_NB = B // _BB
_W = 2 * (K - 1) + 1


def _gather_rows(table_view, idxq):
    mesh = plsc.VectorSubcoreMesh(core_axis_name="c", subcore_axis_name="s")

    @functools.partial(
        pl.kernel,
        mesh=mesh,
        out_type=jax.ShapeDtypeStruct((B, 128), jnp.float32),
        scratch_types=[
            pltpu.VMEM((_BPW,), jnp.int32),
            pltpu.VMEM((_BPW, 128), jnp.float32),
            pltpu.SemaphoreType.DMA,
        ],
    )
    def k(table_hbm, idx_hbm, out_hbm, idx_v, rows_v, sem):
        wid = lax.axis_index("s") * _NC + lax.axis_index("c")
        base = wid * _BPW
        pltpu.sync_copy(idx_hbm.at[pl.ds(base, _BPW)], idx_v)
        pltpu.async_copy(table_hbm.at[idx_v], rows_v, sem).wait()
        pltpu.sync_copy(rows_v, out_hbm.at[pl.ds(base, _BPW)])

    return k(table_view, idxq)


def _encoder_body(pos_ref, phase_ref, gath_ref, hist_ref, w1f_ref, b1_ref,
                  w2f_ref, b2_ref, out_ref):
    pos = pos_ref[...]
    stp = 16 * pos
    a = stp // 256
    rr = stp - 256 * a
    acc = hist_ref[:, 0:416]
    for aa in range(1, 13):
        acc = jnp.where(a == aa, hist_ref[:, 256 * aa:256 * aa + 416], acc)
    c128 = rr >= 128
    r1 = rr - 128 * c128.astype(jnp.int32)
    acc = jnp.where(c128, acc[:, 128:416], acc[:, 0:288])
    c64 = r1 >= 64
    r2 = r1 - 64 * c64.astype(jnp.int32)
    acc = jnp.where(c64, acc[:, 64:288], acc[:, 0:224])
    c32 = r2 >= 32
    r3 = r2 - 32 * c32.astype(jnp.int32)
    acc = jnp.where(c32, acc[:, 32:224], acc[:, 0:192])
    c16 = r3 >= 16
    W = jnp.where(c16, acc[:, 16:192], acc[:, 0:176])
    w1f = w1f_ref[...]
    b1 = b1_ref[...]
    rs = []
    for t in range(K):
        patch = W[:, 16 * t:16 * t + K * C_IN]
        r = jnp.dot(patch, w1f, preferred_element_type=jnp.float32) + b1
        r = jnp.maximum(r, 0.0)
        valid = (pos >= (K - 1) - t).astype(jnp.float32)
        rs.append(r * valid)
    h1 = jnp.concatenate(rs, axis=1)
    enc = jnp.dot(h1, w2f_ref[...], preferred_element_type=jnp.float32)
    enc = enc + b2_ref[...]
    G = gath_ref[...]
    ph = phase_ref[...]
    sub = jnp.zeros((_BB, D_SUB), jnp.float32)
    for c in range(_RPG):
        m = (ph == c).astype(jnp.float32)
        sub = sub + G[:, c * D_SUB:(c + 1) * D_SUB] * m
    n2 = jnp.sum(sub * sub, axis=1, keepdims=True)
    norm = jnp.sqrt(n2)
    scale = jnp.minimum(1.0, MAX_NORM / jnp.maximum(norm, 1e-7))
    out_ref[...] = jnp.concatenate([sub * scale, enc], axis=1)


def kernel(indices, history, history_lengths, table, w1, b1, w2, b2):
    idx = indices.astype(jnp.int32)
    pos2 = jnp.clip(history_lengths.astype(jnp.int32) - 1, 0, L - 1)
    pos2 = pos2.reshape(B, 1)
    table_view = table.reshape(_VG, 128)
    idxq = idx // _RPG
    phase2 = (idx % _RPG).reshape(B, 1)
    gath = _gather_rows(table_view, idxq)

    hist_flat = jnp.pad(
        history.transpose(0, 2, 1).reshape(B, C_IN * L), ((0, 0), (160, 224)))

    w1f = w1.transpose(2, 1, 0).reshape(K * C_IN, H)
    w2f = w2.transpose(2, 1, 0).reshape(K * H, D_ENC)
    b1r = b1.reshape(1, H)
    b2r = b2.reshape(1, D_ENC)

    out = pl.pallas_call(
        _encoder_body,
        grid=(_NB,),
        in_specs=[
            pl.BlockSpec((_BB, 1), lambda i: (i, 0)),
            pl.BlockSpec((_BB, 1), lambda i: (i, 0)),
            pl.BlockSpec((_BB, 128), lambda i: (i, 0)),
            pl.BlockSpec((_BB, 3584), lambda i: (i, 0)),
            pl.BlockSpec((K * C_IN, H), lambda i: (0, 0)),
            pl.BlockSpec((1, H), lambda i: (0, 0)),
            pl.BlockSpec((K * H, D_ENC), lambda i: (0, 0)),
            pl.BlockSpec((1, D_ENC), lambda i: (0, 0)),
        ],
        out_specs=pl.BlockSpec((_BB, D_SUB + D_ENC), lambda i: (i, 0)),
        out_shape=jax.ShapeDtypeStruct((B, D_SUB + D_ENC), jnp.float32),
    )(pos2, phase2, gath, hist_flat, w1f, b1r, w2f, b2r)
    return out


# speedup vs baseline: 1.9753x; 1.1359x over previous
import functools

import jax
import jax.numpy as jnp
from jax import lax
from jax.experimental import pallas as pl
from jax.experimental.pallas import tpu as pltpu
from jax.experimental.pallas import tpu_sc as plsc

B = 4096
V = 100000
D_SUB = 8
D_ENC = 8
C_IN = 16
L = 200
K = 6
H = 128
MAX_NORM = 5.0

_NC = 2
_NS = 16
_NW = _NC * _NS
_BPW = B // _NW

_RPG = 128 // D_SUB
_VG = V * D_SUB // 128

_BB = 128
_NB = B // _BB
_W = 2 * (K - 1) + 1


def _gather_rows(table_view, idxq):
    mesh = plsc.VectorSubcoreMesh(core_axis_name="c", subcore_axis_name="s")

    @functools.partial(
        pl.kernel,
        mesh=mesh,
        out_type=jax.ShapeDtypeStruct((B, 128), jnp.float32),
        scratch_types=[
            pltpu.VMEM((_BPW,), jnp.int32),
            pltpu.VMEM((_BPW, 128), jnp.float32),
            pltpu.SemaphoreType.DMA,
        ],
    )
    def k(table_hbm, idx_hbm, out_hbm, idx_v, rows_v, sem):
        wid = lax.axis_index("s") * _NC + lax.axis_index("c")
        base = wid * _BPW
        pltpu.sync_copy(idx_hbm.at[pl.ds(base, _BPW)], idx_v)
        pltpu.async_copy(table_hbm.at[idx_v], rows_v, sem).wait()
        pltpu.sync_copy(rows_v, out_hbm.at[pl.ds(base, _BPW)])

    return k(table_view, idxq)


def _encoder_body(pos_ref, phase_ref, gath_ref, hist_ref, w1f_ref, b1_ref,
                  w2f_ref, b2_ref, out_ref):
    pos = pos_ref[...]
    stp = 16 * pos
    a = stp // 256
    rr = stp - 256 * a
    acc = hist_ref[:, 0:416]
    for aa in range(1, 13):
        acc = jnp.where(a == aa, hist_ref[:, 256 * aa:256 * aa + 416], acc)
    c128 = rr >= 128
    r1 = rr - 128 * c128.astype(jnp.int32)
    acc = jnp.where(c128, acc[:, 128:416], acc[:, 0:288])
    c64 = r1 >= 64
    r2 = r1 - 64 * c64.astype(jnp.int32)
    acc = jnp.where(c64, acc[:, 64:288], acc[:, 0:224])
    c32 = r2 >= 32
    r3 = r2 - 32 * c32.astype(jnp.int32)
    acc = jnp.where(c32, acc[:, 32:224], acc[:, 0:192])
    c16 = r3 >= 16
    W = jnp.where(c16, acc[:, 16:192], acc[:, 0:176])
    W = W.astype(jnp.float32)
    w1f = w1f_ref[...]
    b1 = b1_ref[...]
    rs = []
    for t in range(K):
        patch = W[:, 16 * t:16 * t + K * C_IN]
        r = jnp.dot(patch, w1f, preferred_element_type=jnp.float32) + b1
        r = jnp.maximum(r, 0.0)
        valid = (pos >= (K - 1) - t).astype(jnp.float32)
        rs.append(r * valid)
    h1 = jnp.concatenate(rs, axis=1)
    enc = jnp.dot(h1, w2f_ref[...], preferred_element_type=jnp.float32)
    enc = enc + b2_ref[...]
    G = gath_ref[...]
    ph = phase_ref[...]
    sub = jnp.zeros((_BB, D_SUB), jnp.float32)
    for c in range(_RPG):
        m = (ph == c).astype(jnp.float32)
        sub = sub + G[:, c * D_SUB:(c + 1) * D_SUB] * m
    n2 = jnp.sum(sub * sub, axis=1, keepdims=True)
    norm = jnp.sqrt(n2)
    scale = jnp.minimum(1.0, MAX_NORM / jnp.maximum(norm, 1e-7))
    out_ref[...] = jnp.concatenate([sub * scale, enc], axis=1)


def kernel(indices, history, history_lengths, table, w1, b1, w2, b2):
    idx = indices.astype(jnp.int32)
    pos2 = jnp.clip(history_lengths.astype(jnp.int32) - 1, 0, L - 1)
    pos2 = pos2.reshape(B, 1)
    table_view = table.reshape(_VG, 128)
    idxq = idx // _RPG
    phase2 = (idx % _RPG).reshape(B, 1)
    gath = _gather_rows(table_view, idxq)

    hist_flat = jnp.pad(
        history.transpose(0, 2, 1).reshape(B, C_IN * L),
        ((0, 0), (160, 224))).astype(jnp.bfloat16)

    w1f = w1.transpose(2, 1, 0).reshape(K * C_IN, H)
    w2f = w2.transpose(2, 1, 0).reshape(K * H, D_ENC)
    b1r = b1.reshape(1, H)
    b2r = b2.reshape(1, D_ENC)

    out = pl.pallas_call(
        _encoder_body,
        grid=(_NB,),
        in_specs=[
            pl.BlockSpec((_BB, 1), lambda i: (i, 0)),
            pl.BlockSpec((_BB, 1), lambda i: (i, 0)),
            pl.BlockSpec((_BB, 128), lambda i: (i, 0)),
            pl.BlockSpec((_BB, 3584), lambda i: (i, 0)),
            pl.BlockSpec((K * C_IN, H), lambda i: (0, 0)),
            pl.BlockSpec((1, H), lambda i: (0, 0)),
            pl.BlockSpec((K * H, D_ENC), lambda i: (0, 0)),
            pl.BlockSpec((1, D_ENC), lambda i: (0, 0)),
        ],
        out_specs=pl.BlockSpec((_BB, D_SUB + D_ENC), lambda i: (i, 0)),
        out_shape=jax.ShapeDtypeStruct((B, D_SUB + D_ENC), jnp.float32),
    )(pos2, phase2, gath, hist_flat, w1f, b1r, w2f, b2r)
    return out
